# baseline (speedup 1.0000x reference)
import functools

import jax
import jax.numpy as jnp
from jax import lax
from jax.experimental import pallas as pl
from jax.experimental.pallas import tpu as pltpu
from jax.experimental.pallas import tpu_sc as plsc

_ROWS = 2 * 8192
_COLS = 4096
_BLOCK_ROWS = 512

_NSC = (_ROWS // 4) * _COLS
_NW = 32
_PER_W = _NSC // _NW
_CHUNK = 16384
_G = _PER_W // _CHUNK
_NBUF = 4

_mesh = plsc.VectorSubcoreMesh(core_axis_name="c", subcore_axis_name="s")


@functools.partial(
    pl.kernel,
    mesh=_mesh,
    out_type=jax.ShapeDtypeStruct((_NSC,), jnp.float32),
    scratch_types=[
        pltpu.VMEM((_NBUF, _CHUNK), jnp.float32),
        pltpu.SemaphoreType.DMA((_NBUF,)),
        pltpu.SemaphoreType.DMA((_NBUF,)),
    ],
)
def _sc_copy(in_hbm, out_hbm, bufs, lsem, ssem):
    wid = lax.axis_index("s") * 2 + lax.axis_index("c")
    base = wid * _PER_W

    for b in range(_NBUF):
        pltpu.make_async_copy(
            in_hbm.at[pl.ds(base + b * _CHUNK, _CHUNK)], bufs.at[b], lsem.at[b]
        ).start()

    def outer(g0, carry):
        for b in range(_NBUF):
            g = g0 * _NBUF + b
            off = base + g * _CHUNK
            pltpu.make_async_copy(
                in_hbm.at[pl.ds(off, _CHUNK)], bufs.at[b], lsem.at[b]
            ).wait()
            pltpu.make_async_copy(
                bufs.at[b], out_hbm.at[pl.ds(off, _CHUNK)], ssem.at[b]
            ).start()

            g2 = g + _NBUF

            @pl.when(g2 < _G)
            def _():
                pltpu.make_async_copy(
                    bufs.at[b], out_hbm.at[pl.ds(off, _CHUNK)], ssem.at[b]
                ).wait()
                pltpu.make_async_copy(
                    in_hbm.at[pl.ds(base + g2 * _CHUNK, _CHUNK)],
                    bufs.at[b],
                    lsem.at[b],
                ).start()

        return carry

    lax.fori_loop(0, _G // _NBUF, outer, 0)

    for b in range(_NBUF):
        off = base + (_G - _NBUF + b) * _CHUNK
        pltpu.make_async_copy(
            bufs.at[b], out_hbm.at[pl.ds(off, _CHUNK)], ssem.at[b]
        ).wait()


def _copy_body(i_ref, o_ref):
    o_ref[...] = i_ref[...]


def kernel(x, bit, alpha):
    del bit, alpha
    x2 = x.reshape(_ROWS, _COLS)
    tc_out = pl.pallas_call(
        _copy_body,
        grid=(_ROWS // _BLOCK_ROWS,),
        in_specs=[pl.BlockSpec((_BLOCK_ROWS, _COLS), lambda i: (i, 0))],
        out_specs=pl.BlockSpec((_BLOCK_ROWS, _COLS), lambda i: (i, 0)),
        out_shape=jax.ShapeDtypeStruct((_ROWS, _COLS), x.dtype),
    )(x2)
    sc_out = _sc_copy(x.reshape(-1)[:_NSC])
    return tc_out.reshape(x.shape), sc_out


# speedup vs baseline: 1.9195x; 1.9195x over previous
import jax
import jax.numpy as jnp
from jax.experimental import pallas as pl
from jax.experimental.pallas import tpu as pltpu

_ROWS = 2 * 8192
_COLS = 4096
_BLOCK_ROWS = 960


def _copy_body(i_ref, o_ref):
    o_ref[...] = i_ref[...]


def kernel(x, bit, alpha):
    del bit, alpha
    x2 = x.reshape(_ROWS, _COLS)
    out = pl.pallas_call(
        _copy_body,
        grid=(-(-_ROWS // _BLOCK_ROWS),),
        in_specs=[pl.BlockSpec((_BLOCK_ROWS, _COLS), lambda i: (i, 0))],
        out_specs=pl.BlockSpec((_BLOCK_ROWS, _COLS), lambda i: (i, 0)),
        out_shape=jax.ShapeDtypeStruct((_ROWS, _COLS), x.dtype),
        compiler_params=pltpu.CompilerParams(skip_device_barrier=True, vmem_limit_bytes=100 * 1024 * 1024),
    )(x2)
    return out.reshape(x.shape)


# speedup vs baseline: 1.9246x; 1.0026x over previous
import jax
import jax.numpy as jnp
from jax.experimental import pallas as pl
from jax.experimental.pallas import tpu as pltpu

_ROWS = 2 * 8192
_COLS = 4096
_BLOCK_ROWS = 1008


def _copy_body(i_ref, o_ref):
    o_ref[...] = i_ref[...]


def kernel(x, bit, alpha):
    del bit, alpha
    x2 = x.reshape(_ROWS, _COLS)
    out = pl.pallas_call(
        _copy_body,
        grid=(-(-_ROWS // _BLOCK_ROWS),),
        in_specs=[pl.BlockSpec((_BLOCK_ROWS, _COLS), lambda i: (i, 0))],
        out_specs=pl.BlockSpec((_BLOCK_ROWS, _COLS), lambda i: (i, 0)),
        out_shape=jax.ShapeDtypeStruct((_ROWS, _COLS), x.dtype),
        compiler_params=pltpu.CompilerParams(skip_device_barrier=True, vmem_limit_bytes=100 * 1024 * 1024),
    )(x2)
    return out.reshape(x.shape)
